# Initial kernel scaffold; baseline (speedup 1.0000x reference)
import jax
import jax.numpy as jnp
from jax.experimental import pallas as pl


def kernel(x, ix, Wf, bf, Wg, bg, Wh, bh):
    raise NotImplementedError("write your pallas kernel here")


# speedup vs baseline: 3.3145x; 3.3145x over previous
import jax
import jax.numpy as jnp
from jax.experimental import pallas as pl
from jax.experimental.pallas import tpu as pltpu

_D = 256
_BN = 256
_N = 160000
_NB = _N // _BN
_S = 10000
_SPAD = 10496
_BM = 656
_BW = _BN + 8


def _ranks(ix, lane, carry_ref, k):
    first = jnp.sum(jnp.where(lane == 0, ix, 0))
    last = jnp.sum(jnp.where(lane == _BN - 1, ix, 0))

    @pl.when(k == 0)
    def _():
        carry_ref[0] = 0
        carry_ref[1] = first

    r_prev = carry_ref[0]
    prev_last = carry_ref[1]
    shifted = jnp.concatenate([jnp.full((1, 1), prev_last, ix.dtype), ix[:, :-1]], axis=1)
    ball = (ix != shifted).astype(jnp.int32)
    b0 = jnp.sum(jnp.where(lane == 0, ball, 0))
    total = jnp.sum(ball)
    iota_s = jax.lax.broadcasted_iota(jnp.int32, (_BN, _BN), 0)
    iota_l = jax.lax.broadcasted_iota(jnp.int32, (_BN, _BN), 1)
    tri = (iota_s <= iota_l).astype(jnp.float32)
    csum = jnp.dot(ball.astype(jnp.float32), tri, preferred_element_type=jnp.float32)
    r_first = r_prev + b0
    base = pl.multiple_of((r_first // 8) * 8, 8)
    c_row = csum.astype(jnp.int32) + (r_first - base - b0)
    carry_ref[0] = r_prev + total
    carry_ref[1] = last
    iota_w = jax.lax.broadcasted_iota(jnp.int32, (_BW, _BN), 0)
    onehot_t = (iota_w == c_row).astype(jnp.float32)
    return base, onehot_t


def _seg_accum_kernel(ix_ref, x_ref, wg_ref, accx_ref, accd_ref, carry_ref):
    k = pl.program_id(0)

    @pl.when(k == 0)
    def _():
        accx_ref[...] = jnp.zeros_like(accx_ref)
        accd_ref[...] = jnp.zeros_like(accd_ref)

    ix = ix_ref[0]
    lane = jax.lax.broadcasted_iota(jnp.int32, (1, _BN), 1)
    base, onehot_t = _ranks(ix, lane, carry_ref, k)

    x = x_ref[...]
    logit = jnp.dot(x, wg_ref[...], preferred_element_type=jnp.float32)
    e128 = jnp.exp(logit)
    vals = x * e128[:, 0:1]
    p = jnp.dot(onehot_t, vals, preferred_element_type=jnp.float32)
    pd = jnp.dot(onehot_t, e128, preferred_element_type=jnp.float32)
    accx_ref[pl.ds(base, _BW), :] += p
    accd_ref[pl.ds(base, _BW), :] += pd


def _mlp_kernel(accx_ref, accd_ref, wf_ref, bf_ref, wh_ref, bh_ref, hy_ref):
    d = accd_ref[:, 0:1]
    t = accx_ref[...] / jnp.where(d > 0, d, 1.0)
    y = jax.lax.dot_general(t, wf_ref[...], (((1,), (1,)), ((), ())),
                            preferred_element_type=jnp.float32) + bf_ref[...]
    hy_ref[...] = jax.lax.dot_general(y, wh_ref[...], (((1,), (1,)), ((), ())),
                                      preferred_element_type=jnp.float32) + bh_ref[...]


def _expand_kernel(ix_ref, hy_ref, out_ref, carry_ref):
    k = pl.program_id(0)
    ix = ix_ref[0]
    lane = jax.lax.broadcasted_iota(jnp.int32, (1, _BN), 1)
    base, onehot_t = _ranks(ix, lane, carry_ref, k)
    win = hy_ref[pl.ds(base, _BW), :]
    out_ref[...] = jax.lax.dot_general(onehot_t, win, (((0,), (0,)), ((), ())),
                                       preferred_element_type=jnp.float32)


def kernel(x, ix, Wf, bf, Wg, bg, Wh, bh):
    x2 = x.reshape(_N, _D)
    ix3 = ix.astype(jnp.int32).reshape(_NB, 1, _BN)
    wg128 = jnp.broadcast_to(Wg.reshape(_D, 1), (_D, 128))

    accx, accd = pl.pallas_call(
        _seg_accum_kernel,
        grid=(_NB,),
        in_specs=[
            pl.BlockSpec((1, 1, _BN), lambda k: (k, 0, 0)),
            pl.BlockSpec((_BN, _D), lambda k: (k, 0)),
            pl.BlockSpec((_D, 128), lambda k: (0, 0)),
        ],
        out_specs=[
            pl.BlockSpec((_SPAD, _D), lambda k: (0, 0)),
            pl.BlockSpec((_SPAD, 128), lambda k: (0, 0)),
        ],
        out_shape=[
            jax.ShapeDtypeStruct((_SPAD, _D), jnp.float32),
            jax.ShapeDtypeStruct((_SPAD, 128), jnp.float32),
        ],
        scratch_shapes=[pltpu.SMEM((2,), jnp.int32)],
    )(ix3, x2, wg128)

    hy = pl.pallas_call(
        _mlp_kernel,
        grid=(_SPAD // _BM,),
        in_specs=[
            pl.BlockSpec((_BM, _D), lambda k: (k, 0)),
            pl.BlockSpec((_BM, 128), lambda k: (k, 0)),
            pl.BlockSpec((_D, _D), lambda k: (0, 0)),
            pl.BlockSpec((1, _D), lambda k: (0, 0)),
            pl.BlockSpec((_D, _D), lambda k: (0, 0)),
            pl.BlockSpec((1, _D), lambda k: (0, 0)),
        ],
        out_specs=pl.BlockSpec((_BM, _D), lambda k: (k, 0)),
        out_shape=jax.ShapeDtypeStruct((_SPAD, _D), jnp.float32),
    )(accx, accd, Wf, bf.reshape(1, _D), Wh, bh.reshape(1, _D))

    out = pl.pallas_call(
        _expand_kernel,
        grid=(_NB,),
        in_specs=[
            pl.BlockSpec((1, 1, _BN), lambda k: (k, 0, 0)),
            pl.BlockSpec((_SPAD, _D), lambda k: (0, 0)),
        ],
        out_specs=pl.BlockSpec((_BN, _D), lambda k: (k, 0)),
        out_shape=jax.ShapeDtypeStruct((_N, _D), jnp.float32),
        scratch_shapes=[pltpu.SMEM((2,), jnp.int32)],
    )(ix3, hy)

    return out.reshape(1, _N, _D)


# speedup vs baseline: 3.3151x; 1.0002x over previous
import jax
import jax.numpy as jnp
from jax.experimental import pallas as pl
from jax.experimental.pallas import tpu as pltpu

_D = 256
_BN = 256
_N = 160000
_NB = _N // _BN
_S = 10000
_SPAD = 10496
_BM = 656
_BW = _BN + 8


def _ranks(ix, lane, carry_ref, k):
    first = jnp.sum(jnp.where(lane == 0, ix, 0))
    last = jnp.sum(jnp.where(lane == _BN - 1, ix, 0))

    @pl.when(k == 0)
    def _():
        carry_ref[0] = 0
        carry_ref[1] = first

    r_prev = carry_ref[0]
    prev_last = carry_ref[1]
    shifted = jnp.concatenate([jnp.full((1, 1), prev_last, ix.dtype), ix[:, :-1]], axis=1)
    ball = (ix != shifted).astype(jnp.int32)
    b0 = jnp.sum(jnp.where(lane == 0, ball, 0))
    total = jnp.sum(ball)
    iota_s = jax.lax.broadcasted_iota(jnp.int32, (_BN, _BN), 0)
    iota_l = jax.lax.broadcasted_iota(jnp.int32, (_BN, _BN), 1)
    tri = (iota_s <= iota_l).astype(jnp.float32)
    csum = jnp.dot(ball.astype(jnp.float32), tri, preferred_element_type=jnp.float32)
    r_first = r_prev + b0
    base = pl.multiple_of((r_first // 8) * 8, 8)
    c_row = csum.astype(jnp.int32) + (r_first - base - b0)
    carry_ref[0] = r_prev + total
    carry_ref[1] = last
    iota_w = jax.lax.broadcasted_iota(jnp.int32, (_BW, _BN), 0)
    onehot_t = (iota_w == c_row).astype(jnp.bfloat16)
    return base, onehot_t


def _seg_accum_kernel(ix_ref, x_ref, wg_ref, accx_ref, accd_ref, carry_ref):
    k = pl.program_id(0)

    @pl.when(k == 0)
    def _():
        accx_ref[...] = jnp.zeros_like(accx_ref)
        accd_ref[...] = jnp.zeros_like(accd_ref)

    ix = ix_ref[0]
    lane = jax.lax.broadcasted_iota(jnp.int32, (1, _BN), 1)
    base, onehot_t = _ranks(ix, lane, carry_ref, k)

    x = x_ref[...]
    logit = jnp.dot(x, wg_ref[...], preferred_element_type=jnp.float32)
    e128 = jnp.exp(logit)
    vals = (x * e128[:, 0:1]).astype(jnp.bfloat16)
    p = jnp.dot(onehot_t, vals, preferred_element_type=jnp.float32)
    pd = jnp.dot(onehot_t, e128.astype(jnp.bfloat16),
                 preferred_element_type=jnp.float32)
    accx_ref[pl.ds(base, _BW), :] += p
    accd_ref[pl.ds(base, _BW), :] += pd


def _mlp_kernel(accx_ref, accd_ref, wf_ref, bf_ref, wh_ref, bh_ref, hy_ref):
    d = accd_ref[:, 0:1]
    t = accx_ref[...] / jnp.where(d > 0, d, 1.0)
    y = jax.lax.dot_general(t, wf_ref[...], (((1,), (1,)), ((), ())),
                            preferred_element_type=jnp.float32) + bf_ref[...]
    hy_ref[...] = jax.lax.dot_general(y, wh_ref[...], (((1,), (1,)), ((), ())),
                                      preferred_element_type=jnp.float32) + bh_ref[...]


def _expand_kernel(ix_ref, hy_ref, out_ref, carry_ref):
    k = pl.program_id(0)
    ix = ix_ref[0]
    lane = jax.lax.broadcasted_iota(jnp.int32, (1, _BN), 1)
    base, onehot_t = _ranks(ix, lane, carry_ref, k)
    win = hy_ref[pl.ds(base, _BW), :].astype(jnp.bfloat16)
    out_ref[...] = jax.lax.dot_general(onehot_t, win, (((0,), (0,)), ((), ())),
                                       preferred_element_type=jnp.float32)


def kernel(x, ix, Wf, bf, Wg, bg, Wh, bh):
    x2 = x.reshape(_N, _D)
    ix3 = ix.astype(jnp.int32).reshape(_NB, 1, _BN)
    wg128 = jnp.broadcast_to(Wg.reshape(_D, 1), (_D, 128))

    accx, accd = pl.pallas_call(
        _seg_accum_kernel,
        grid=(_NB,),
        in_specs=[
            pl.BlockSpec((1, 1, _BN), lambda k: (k, 0, 0)),
            pl.BlockSpec((_BN, _D), lambda k: (k, 0)),
            pl.BlockSpec((_D, 128), lambda k: (0, 0)),
        ],
        out_specs=[
            pl.BlockSpec((_SPAD, _D), lambda k: (0, 0)),
            pl.BlockSpec((_SPAD, 128), lambda k: (0, 0)),
        ],
        out_shape=[
            jax.ShapeDtypeStruct((_SPAD, _D), jnp.float32),
            jax.ShapeDtypeStruct((_SPAD, 128), jnp.float32),
        ],
        scratch_shapes=[pltpu.SMEM((2,), jnp.int32)],
    )(ix3, x2, wg128)

    hy = pl.pallas_call(
        _mlp_kernel,
        grid=(_SPAD // _BM,),
        in_specs=[
            pl.BlockSpec((_BM, _D), lambda k: (k, 0)),
            pl.BlockSpec((_BM, 128), lambda k: (k, 0)),
            pl.BlockSpec((_D, _D), lambda k: (0, 0)),
            pl.BlockSpec((1, _D), lambda k: (0, 0)),
            pl.BlockSpec((_D, _D), lambda k: (0, 0)),
            pl.BlockSpec((1, _D), lambda k: (0, 0)),
        ],
        out_specs=pl.BlockSpec((_BM, _D), lambda k: (k, 0)),
        out_shape=jax.ShapeDtypeStruct((_SPAD, _D), jnp.float32),
    )(accx, accd, Wf, bf.reshape(1, _D), Wh, bh.reshape(1, _D))

    out = pl.pallas_call(
        _expand_kernel,
        grid=(_NB,),
        in_specs=[
            pl.BlockSpec((1, 1, _BN), lambda k: (k, 0, 0)),
            pl.BlockSpec((_SPAD, _D), lambda k: (0, 0)),
        ],
        out_specs=pl.BlockSpec((_BN, _D), lambda k: (k, 0)),
        out_shape=jax.ShapeDtypeStruct((_N, _D), jnp.float32),
        scratch_shapes=[pltpu.SMEM((2,), jnp.int32)],
    )(ix3, hy)

    return out.reshape(1, _N, _D)


# speedup vs baseline: 3.3667x; 1.0156x over previous
import numpy as np
import jax
import jax.numpy as jnp
from jax.experimental import pallas as pl
from jax.experimental.pallas import tpu as pltpu

_D = 256
_BN = 256
_N = 160000
_NB = _N // _BN
_S = 10000
_SPAD = 10496
_BM = 656
_BW = _BN + 8
_WF = 64

_TRI = np.triu(np.ones((_BN, _BN), np.float32))


def _ranks(ix, lane, tri_ref, carry_ref, k):
    first = jnp.sum(jnp.where(lane == 0, ix, 0))
    last = jnp.sum(jnp.where(lane == _BN - 1, ix, 0))

    @pl.when(k == 0)
    def _():
        carry_ref[0] = 0
        carry_ref[1] = first

    r_prev = carry_ref[0]
    prev_last = carry_ref[1]
    shifted = jnp.concatenate([jnp.full((1, 1), prev_last, ix.dtype), ix[:, :-1]], axis=1)
    ball = (ix != shifted).astype(jnp.int32)
    b0 = jnp.sum(jnp.where(lane == 0, ball, 0))
    total = jnp.sum(ball)
    csum = jnp.dot(ball.astype(jnp.float32), tri_ref[...],
                   preferred_element_type=jnp.float32)
    r_first = r_prev + b0
    base = pl.multiple_of((r_first // 8) * 8, 8)
    c_row = csum.astype(jnp.int32) + (r_first - base - b0)
    nwin = r_first - base + total + 1
    carry_ref[0] = r_prev + total
    carry_ref[1] = last
    return base, c_row, nwin


def _onehot_t(c_row, w):
    iota_w = jax.lax.broadcasted_iota(jnp.int32, (w, _BN), 0)
    return (iota_w == c_row).astype(jnp.bfloat16)


def _seg_accum_kernel(ix_ref, x_ref, wg_ref, tri_ref, accx_ref, accd_ref, carry_ref):
    k = pl.program_id(0)

    @pl.when(k == 0)
    def _():
        accx_ref[...] = jnp.zeros_like(accx_ref)
        accd_ref[...] = jnp.zeros_like(accd_ref)

    ix = ix_ref[0]
    lane = jax.lax.broadcasted_iota(jnp.int32, (1, _BN), 1)
    base, c_row, nwin = _ranks(ix, lane, tri_ref, carry_ref, k)

    x = x_ref[...]
    logit = jnp.dot(x.astype(jnp.bfloat16), wg_ref[...],
                    preferred_element_type=jnp.float32)
    e128 = jnp.exp(logit)
    vals = (x * e128[:, 0:1]).astype(jnp.bfloat16)
    e128b = e128.astype(jnp.bfloat16)

    @pl.when(nwin <= _WF)
    def _():
        oh = _onehot_t(c_row, _WF)
        accx_ref[pl.ds(base, _WF), :] += jnp.dot(oh, vals, preferred_element_type=jnp.float32)
        accd_ref[pl.ds(base, _WF), :] += jnp.dot(oh, e128b, preferred_element_type=jnp.float32)

    @pl.when(nwin > _WF)
    def _():
        oh = _onehot_t(c_row, _BW)
        accx_ref[pl.ds(base, _BW), :] += jnp.dot(oh, vals, preferred_element_type=jnp.float32)
        accd_ref[pl.ds(base, _BW), :] += jnp.dot(oh, e128b, preferred_element_type=jnp.float32)


def _mlp_kernel(accx_ref, accd_ref, wf_ref, bf_ref, wh_ref, bh_ref, hy_ref):
    d = accd_ref[:, 0:1]
    t = accx_ref[...] / jnp.where(d > 0, d, 1.0)
    y = jax.lax.dot_general(t, wf_ref[...], (((1,), (1,)), ((), ())),
                            preferred_element_type=jnp.float32) + bf_ref[...]
    hy_ref[...] = jax.lax.dot_general(y, wh_ref[...], (((1,), (1,)), ((), ())),
                                      preferred_element_type=jnp.float32) + bh_ref[...]


def _expand_kernel(ix_ref, hy_ref, tri_ref, out_ref, carry_ref):
    k = pl.program_id(0)
    ix = ix_ref[0]
    lane = jax.lax.broadcasted_iota(jnp.int32, (1, _BN), 1)
    base, c_row, nwin = _ranks(ix, lane, tri_ref, carry_ref, k)

    @pl.when(nwin <= _WF)
    def _():
        win = hy_ref[pl.ds(base, _WF), :].astype(jnp.bfloat16)
        out_ref[...] = jax.lax.dot_general(
            _onehot_t(c_row, _WF), win, (((0,), (0,)), ((), ())),
            preferred_element_type=jnp.float32)

    @pl.when(nwin > _WF)
    def _():
        win = hy_ref[pl.ds(base, _BW), :].astype(jnp.bfloat16)
        out_ref[...] = jax.lax.dot_general(
            _onehot_t(c_row, _BW), win, (((0,), (0,)), ((), ())),
            preferred_element_type=jnp.float32)


def kernel(x, ix, Wf, bf, Wg, bg, Wh, bh):
    x2 = x.reshape(_N, _D)
    ix3 = ix.astype(jnp.int32).reshape(_NB, 1, _BN)
    wg128 = jnp.broadcast_to(Wg.reshape(_D, 1), (_D, 128)).astype(jnp.bfloat16)
    tri = jnp.asarray(_TRI)

    accx, accd = pl.pallas_call(
        _seg_accum_kernel,
        grid=(_NB,),
        in_specs=[
            pl.BlockSpec((1, 1, _BN), lambda k: (k, 0, 0)),
            pl.BlockSpec((_BN, _D), lambda k: (k, 0)),
            pl.BlockSpec((_D, 128), lambda k: (0, 0)),
            pl.BlockSpec((_BN, _BN), lambda k: (0, 0)),
        ],
        out_specs=[
            pl.BlockSpec((_SPAD, _D), lambda k: (0, 0)),
            pl.BlockSpec((_SPAD, 128), lambda k: (0, 0)),
        ],
        out_shape=[
            jax.ShapeDtypeStruct((_SPAD, _D), jnp.float32),
            jax.ShapeDtypeStruct((_SPAD, 128), jnp.float32),
        ],
        scratch_shapes=[pltpu.SMEM((2,), jnp.int32)],
    )(ix3, x2, wg128, tri)

    hy = pl.pallas_call(
        _mlp_kernel,
        grid=(_SPAD // _BM,),
        in_specs=[
            pl.BlockSpec((_BM, _D), lambda k: (k, 0)),
            pl.BlockSpec((_BM, 128), lambda k: (k, 0)),
            pl.BlockSpec((_D, _D), lambda k: (0, 0)),
            pl.BlockSpec((1, _D), lambda k: (0, 0)),
            pl.BlockSpec((_D, _D), lambda k: (0, 0)),
            pl.BlockSpec((1, _D), lambda k: (0, 0)),
        ],
        out_specs=pl.BlockSpec((_BM, _D), lambda k: (k, 0)),
        out_shape=jax.ShapeDtypeStruct((_SPAD, _D), jnp.float32),
    )(accx, accd, Wf, bf.reshape(1, _D), Wh, bh.reshape(1, _D))

    out = pl.pallas_call(
        _expand_kernel,
        grid=(_NB,),
        in_specs=[
            pl.BlockSpec((1, 1, _BN), lambda k: (k, 0, 0)),
            pl.BlockSpec((_SPAD, _D), lambda k: (0, 0)),
            pl.BlockSpec((_BN, _BN), lambda k: (0, 0)),
        ],
        out_specs=pl.BlockSpec((_BN, _D), lambda k: (k, 0)),
        out_shape=jax.ShapeDtypeStruct((_N, _D), jnp.float32),
        scratch_shapes=[pltpu.SMEM((2,), jnp.int32)],
    )(ix3, hy, tri)

    return out.reshape(1, _N, _D)


# speedup vs baseline: 8.7276x; 2.5923x over previous
import numpy as np
import jax
import jax.numpy as jnp
from jax.experimental import pallas as pl
from jax.experimental.pallas import tpu as pltpu

_D = 256
_BN = 1280
_N = 160000
_NB = _N // _BN
_S = 10000
_SPAD = 11392
_BM = 712
_BW = _BN + 8
_WF = 160

_TRI = np.triu(np.ones((_BN, _BN), np.float32))


def _ranks(ix, lane, tri_ref, carry_ref, k):
    first = jnp.sum(jnp.where(lane == 0, ix, 0))
    last = jnp.sum(jnp.where(lane == _BN - 1, ix, 0))

    @pl.when(k == 0)
    def _():
        carry_ref[0] = 0
        carry_ref[1] = first

    r_prev = carry_ref[0]
    prev_last = carry_ref[1]
    shifted = jnp.concatenate([jnp.full((1, 1), prev_last, ix.dtype), ix[:, :-1]], axis=1)
    ball = (ix != shifted).astype(jnp.int32)
    b0 = jnp.sum(jnp.where(lane == 0, ball, 0))
    total = jnp.sum(ball)
    csum = jnp.dot(ball.astype(jnp.float32), tri_ref[...],
                   preferred_element_type=jnp.float32)
    r_first = r_prev + b0
    base = pl.multiple_of((r_first // 8) * 8, 8)
    c_row = csum.astype(jnp.int32) + (r_first - base - b0)
    nwin = r_first - base + total + 1
    carry_ref[0] = r_prev + total
    carry_ref[1] = last
    return base, c_row, nwin


def _onehot_t(c_row, w):
    iota_w = jax.lax.broadcasted_iota(jnp.int32, (w, _BN), 0)
    return (iota_w == c_row).astype(jnp.bfloat16)


def _seg_accum_kernel(ix_ref, x_ref, wg_ref, tri_ref, accx_ref, accd_ref, carry_ref):
    k = pl.program_id(0)

    @pl.when(k == 0)
    def _():
        accx_ref[...] = jnp.zeros_like(accx_ref)
        accd_ref[...] = jnp.zeros_like(accd_ref)

    ix = ix_ref[0]
    lane = jax.lax.broadcasted_iota(jnp.int32, (1, _BN), 1)
    base, c_row, nwin = _ranks(ix, lane, tri_ref, carry_ref, k)

    x = x_ref[...]
    logit = jnp.dot(x.astype(jnp.bfloat16), wg_ref[...],
                    preferred_element_type=jnp.float32)
    e128 = jnp.exp(logit)
    vals = (x * e128[:, 0:1]).astype(jnp.bfloat16)
    e128b = e128.astype(jnp.bfloat16)

    @pl.when(nwin <= _WF)
    def _():
        oh = _onehot_t(c_row, _WF)
        accx_ref[pl.ds(base, _WF), :] += jnp.dot(oh, vals, preferred_element_type=jnp.float32)
        accd_ref[pl.ds(base, _WF), :] += jnp.dot(oh, e128b, preferred_element_type=jnp.float32)

    @pl.when(nwin > _WF)
    def _():
        oh = _onehot_t(c_row, _BW)
        accx_ref[pl.ds(base, _BW), :] += jnp.dot(oh, vals, preferred_element_type=jnp.float32)
        accd_ref[pl.ds(base, _BW), :] += jnp.dot(oh, e128b, preferred_element_type=jnp.float32)


def _mlp_kernel(accx_ref, accd_ref, wf_ref, bf_ref, wh_ref, bh_ref, hy_ref):
    d = accd_ref[:, 0:1]
    t = accx_ref[...] / jnp.where(d > 0, d, 1.0)
    y = jax.lax.dot_general(t, wf_ref[...], (((1,), (1,)), ((), ())),
                            preferred_element_type=jnp.float32) + bf_ref[...]
    hy_ref[...] = jax.lax.dot_general(y, wh_ref[...], (((1,), (1,)), ((), ())),
                                      preferred_element_type=jnp.float32) + bh_ref[...]


def _expand_kernel(ix_ref, hy_ref, tri_ref, out_ref, carry_ref):
    k = pl.program_id(0)
    ix = ix_ref[0]
    lane = jax.lax.broadcasted_iota(jnp.int32, (1, _BN), 1)
    base, c_row, nwin = _ranks(ix, lane, tri_ref, carry_ref, k)

    @pl.when(nwin <= _WF)
    def _():
        win = hy_ref[pl.ds(base, _WF), :].astype(jnp.bfloat16)
        out_ref[...] = jax.lax.dot_general(
            _onehot_t(c_row, _WF), win, (((0,), (0,)), ((), ())),
            preferred_element_type=jnp.float32)

    @pl.when(nwin > _WF)
    def _():
        win = hy_ref[pl.ds(base, _BW), :].astype(jnp.bfloat16)
        out_ref[...] = jax.lax.dot_general(
            _onehot_t(c_row, _BW), win, (((0,), (0,)), ((), ())),
            preferred_element_type=jnp.float32)


def kernel(x, ix, Wf, bf, Wg, bg, Wh, bh):
    x2 = x.reshape(_N, _D)
    ix3 = ix.astype(jnp.int32).reshape(_NB, 1, _BN)
    wg128 = jnp.broadcast_to(Wg.reshape(_D, 1), (_D, 128)).astype(jnp.bfloat16)
    tri = jnp.asarray(_TRI)

    accx, accd = pl.pallas_call(
        _seg_accum_kernel,
        grid=(_NB,),
        in_specs=[
            pl.BlockSpec((1, 1, _BN), lambda k: (k, 0, 0)),
            pl.BlockSpec((_BN, _D), lambda k: (k, 0)),
            pl.BlockSpec((_D, 128), lambda k: (0, 0)),
            pl.BlockSpec((_BN, _BN), lambda k: (0, 0)),
        ],
        out_specs=[
            pl.BlockSpec((_SPAD, _D), lambda k: (0, 0)),
            pl.BlockSpec((_SPAD, 128), lambda k: (0, 0)),
        ],
        out_shape=[
            jax.ShapeDtypeStruct((_SPAD, _D), jnp.float32),
            jax.ShapeDtypeStruct((_SPAD, 128), jnp.float32),
        ],
        scratch_shapes=[pltpu.SMEM((2,), jnp.int32)],
    )(ix3, x2, wg128, tri)

    hy = pl.pallas_call(
        _mlp_kernel,
        grid=(_SPAD // _BM,),
        in_specs=[
            pl.BlockSpec((_BM, _D), lambda k: (k, 0)),
            pl.BlockSpec((_BM, 128), lambda k: (k, 0)),
            pl.BlockSpec((_D, _D), lambda k: (0, 0)),
            pl.BlockSpec((1, _D), lambda k: (0, 0)),
            pl.BlockSpec((_D, _D), lambda k: (0, 0)),
            pl.BlockSpec((1, _D), lambda k: (0, 0)),
        ],
        out_specs=pl.BlockSpec((_BM, _D), lambda k: (k, 0)),
        out_shape=jax.ShapeDtypeStruct((_SPAD, _D), jnp.float32),
    )(accx, accd, Wf, bf.reshape(1, _D), Wh, bh.reshape(1, _D))

    out = pl.pallas_call(
        _expand_kernel,
        grid=(_NB,),
        in_specs=[
            pl.BlockSpec((1, 1, _BN), lambda k: (k, 0, 0)),
            pl.BlockSpec((_SPAD, _D), lambda k: (0, 0)),
            pl.BlockSpec((_BN, _BN), lambda k: (0, 0)),
        ],
        out_specs=pl.BlockSpec((_BN, _D), lambda k: (k, 0)),
        out_shape=jax.ShapeDtypeStruct((_N, _D), jnp.float32),
        scratch_shapes=[pltpu.SMEM((2,), jnp.int32)],
    )(ix3, hy, tri)

    return out.reshape(1, _N, _D)


# speedup vs baseline: 9.5506x; 1.0943x over previous
import numpy as np
import jax
import jax.numpy as jnp
from jax.experimental import pallas as pl
from jax.experimental.pallas import tpu as pltpu

_D = 256
_BN = 1280
_N = 160000
_NB = _N // _BN
_S = 10000
_SPAD = 11392
_BM = 712
_BW = _BN + 8
_WF = 160

_TRI = np.triu(np.ones((_BN, _BN), np.float32))


def _ranks(ix, lane, tri_ref, carry_ref, k):
    first = jnp.sum(jnp.where(lane == 0, ix, 0))
    last = jnp.sum(jnp.where(lane == _BN - 1, ix, 0))

    @pl.when(k == 0)
    def _():
        carry_ref[0] = 0
        carry_ref[1] = first

    r_prev = carry_ref[0]
    prev_last = carry_ref[1]
    shifted = jnp.concatenate([jnp.full((1, 1), prev_last, ix.dtype), ix[:, :-1]], axis=1)
    ball = (ix != shifted).astype(jnp.int32)
    b0 = jnp.sum(jnp.where(lane == 0, ball, 0))
    total = jnp.sum(ball)
    csum = jnp.dot(ball.astype(jnp.bfloat16), tri_ref[...],
                   preferred_element_type=jnp.float32)
    r_first = r_prev + b0
    base = pl.multiple_of((r_first // 8) * 8, 8)
    c_row = csum.astype(jnp.int32) + (r_first - base - b0)
    nwin = r_first - base + total + 1
    carry_ref[0] = r_prev + total
    carry_ref[1] = last
    return base, c_row, nwin


def _onehot_t(c_row, w):
    iota_w = jax.lax.broadcasted_iota(jnp.int32, (w, _BN), 0)
    return (iota_w == c_row).astype(jnp.bfloat16)


def _seg_accum_kernel(ix_ref, x_ref, wg_ref, tri_ref, accx_ref, accd_ref, r_ref,
                      carry_ref):
    k = pl.program_id(0)

    @pl.when(k == 0)
    def _():
        accx_ref[...] = jnp.zeros_like(accx_ref)
        accd_ref[...] = jnp.zeros_like(accd_ref)

    ix = ix_ref[0]
    lane = jax.lax.broadcasted_iota(jnp.int32, (1, _BN), 1)
    base, c_row, nwin = _ranks(ix, lane, tri_ref, carry_ref, k)
    r_ref[0] = c_row + base

    x = x_ref[...]
    logit = jnp.dot(x.astype(jnp.bfloat16), wg_ref[...],
                    preferred_element_type=jnp.float32)
    e128 = jnp.exp(logit)
    vals = (x * e128[:, 0:1]).astype(jnp.bfloat16)
    e128b = e128.astype(jnp.bfloat16)

    @pl.when(nwin <= _WF)
    def _():
        oh = _onehot_t(c_row, _WF)
        accx_ref[pl.ds(base, _WF), :] += jnp.dot(oh, vals, preferred_element_type=jnp.float32)
        accd_ref[pl.ds(base, _WF), :] += jnp.dot(oh, e128b, preferred_element_type=jnp.float32)

    @pl.when(nwin > _WF)
    def _():
        oh = _onehot_t(c_row, _BW)
        accx_ref[pl.ds(base, _BW), :] += jnp.dot(oh, vals, preferred_element_type=jnp.float32)
        accd_ref[pl.ds(base, _BW), :] += jnp.dot(oh, e128b, preferred_element_type=jnp.float32)


def _mlp_kernel(accx_ref, accd_ref, wf_ref, bf_ref, wh_ref, bh_ref, hy_ref):
    d = accd_ref[:, 0:1]
    t = accx_ref[...] / jnp.where(d > 0, d, 1.0)
    y = jax.lax.dot_general(t, wf_ref[...], (((1,), (1,)), ((), ())),
                            preferred_element_type=jnp.float32) + bf_ref[...]
    hy_ref[...] = jax.lax.dot_general(y, wh_ref[...], (((1,), (1,)), ((), ())),
                                      preferred_element_type=jnp.float32) + bh_ref[...]


def _expand_kernel(r_ref, hy_ref, out_ref):
    r = r_ref[0]
    lane = jax.lax.broadcasted_iota(jnp.int32, (1, _BN), 1)
    r0 = jnp.sum(jnp.where(lane == 0, r, 0))
    rlast = jnp.sum(jnp.where(lane == _BN - 1, r, 0))
    base = pl.multiple_of((r0 // 8) * 8, 8)
    c_row = r - base
    nwin = rlast - base + 1

    @pl.when(nwin <= _WF)
    def _():
        win = hy_ref[pl.ds(base, _WF), :].astype(jnp.bfloat16)
        out_ref[...] = jax.lax.dot_general(
            _onehot_t(c_row, _WF), win, (((0,), (0,)), ((), ())),
            preferred_element_type=jnp.float32)

    @pl.when(nwin > _WF)
    def _():
        win = hy_ref[pl.ds(base, _BW), :].astype(jnp.bfloat16)
        out_ref[...] = jax.lax.dot_general(
            _onehot_t(c_row, _BW), win, (((0,), (0,)), ((), ())),
            preferred_element_type=jnp.float32)


def kernel(x, ix, Wf, bf, Wg, bg, Wh, bh):
    x2 = x.reshape(_N, _D)
    ix3 = ix.astype(jnp.int32).reshape(_NB, 1, _BN)
    wg128 = jnp.broadcast_to(Wg.reshape(_D, 1), (_D, 128)).astype(jnp.bfloat16)
    tri = jnp.asarray(_TRI).astype(jnp.bfloat16)

    accx, accd, rks = pl.pallas_call(
        _seg_accum_kernel,
        grid=(_NB,),
        in_specs=[
            pl.BlockSpec((1, 1, _BN), lambda k: (k, 0, 0)),
            pl.BlockSpec((_BN, _D), lambda k: (k, 0)),
            pl.BlockSpec((_D, 128), lambda k: (0, 0)),
            pl.BlockSpec((_BN, _BN), lambda k: (0, 0)),
        ],
        out_specs=[
            pl.BlockSpec((_SPAD, _D), lambda k: (0, 0)),
            pl.BlockSpec((_SPAD, 128), lambda k: (0, 0)),
            pl.BlockSpec((1, 1, _BN), lambda k: (k, 0, 0)),
        ],
        out_shape=[
            jax.ShapeDtypeStruct((_SPAD, _D), jnp.float32),
            jax.ShapeDtypeStruct((_SPAD, 128), jnp.float32),
            jax.ShapeDtypeStruct((_NB, 1, _BN), jnp.int32),
        ],
        scratch_shapes=[pltpu.SMEM((2,), jnp.int32)],
    )(ix3, x2, wg128, tri)

    hy = pl.pallas_call(
        _mlp_kernel,
        grid=(_SPAD // _BM,),
        in_specs=[
            pl.BlockSpec((_BM, _D), lambda k: (k, 0)),
            pl.BlockSpec((_BM, 128), lambda k: (k, 0)),
            pl.BlockSpec((_D, _D), lambda k: (0, 0)),
            pl.BlockSpec((1, _D), lambda k: (0, 0)),
            pl.BlockSpec((_D, _D), lambda k: (0, 0)),
            pl.BlockSpec((1, _D), lambda k: (0, 0)),
        ],
        out_specs=pl.BlockSpec((_BM, _D), lambda k: (k, 0)),
        out_shape=jax.ShapeDtypeStruct((_SPAD, _D), jnp.float32),
    )(accx, accd, Wf, bf.reshape(1, _D), Wh, bh.reshape(1, _D))

    out = pl.pallas_call(
        _expand_kernel,
        grid=(_NB,),
        in_specs=[
            pl.BlockSpec((1, 1, _BN), lambda k: (k, 0, 0)),
            pl.BlockSpec((_SPAD, _D), lambda k: (0, 0)),
        ],
        out_specs=pl.BlockSpec((_BN, _D), lambda k: (k, 0)),
        out_shape=jax.ShapeDtypeStruct((_N, _D), jnp.float32),
    )(rks, hy)

    return out.reshape(1, _N, _D)


# speedup vs baseline: 10.5149x; 1.1010x over previous
import numpy as np
import jax
import jax.numpy as jnp
from jax.experimental import pallas as pl
from jax.experimental.pallas import tpu as pltpu

_D = 256
_BN = 1280
_N = 160000
_NB = _N // _BN
_S = 10000
_SPAD = 11392
_BM = 712
_BW = _BN + 8
_WF = 128
_NR = _BN // 128

_TRI = np.triu(np.ones((128, 128), np.float32))
_TRIS = np.tril(np.ones((16, 16), np.float32), -1)


def _ranks(ix, lane, tri_ref, tris_ref, carry_ref, k):
    first = jnp.sum(jnp.where(lane == 0, ix, 0))
    last = jnp.sum(jnp.where(lane == _BN - 1, ix, 0))

    @pl.when(k == 0)
    def _():
        carry_ref[0] = 0
        carry_ref[1] = first

    r_prev = carry_ref[0]
    prev_last = carry_ref[1]
    shifted = jnp.concatenate([jnp.full((1, 1), prev_last, ix.dtype), ix[:, :-1]], axis=1)
    ball = (ix != shifted).astype(jnp.int32)
    b0 = jnp.sum(jnp.where(lane == 0, ball, 0))
    total = jnp.sum(ball)
    ball2 = jnp.concatenate(
        [ball.astype(jnp.bfloat16).reshape(_NR, 128),
         jnp.zeros((16 - _NR, 128), jnp.bfloat16)], axis=0)
    csum2 = jnp.dot(ball2, tri_ref[...], preferred_element_type=jnp.float32)
    t = jnp.dot(tris_ref[...], ball2, preferred_element_type=jnp.float32)
    pre = jnp.sum(t, axis=1, keepdims=True)
    csum = (csum2 + pre)[:_NR].reshape(1, _BN)
    r_first = r_prev + b0
    base = pl.multiple_of((r_first // 8) * 8, 8)
    c_row = csum.astype(jnp.int32) + (r_first - base - b0)
    nwin = r_first - base + total + 1
    carry_ref[0] = r_prev + total
    carry_ref[1] = last
    return base, c_row, nwin


def _onehot_t(c_row, w):
    iota_w = jax.lax.broadcasted_iota(jnp.int32, (w, _BN), 0)
    return (iota_w == c_row).astype(jnp.bfloat16)


def _seg_accum_kernel(ix_ref, x_ref, wg_ref, tri_ref, tris_ref, accx_ref,
                      accd_ref, r_ref, carry_ref):
    k = pl.program_id(0)

    @pl.when(k == 0)
    def _():
        accx_ref[...] = jnp.zeros_like(accx_ref)
        accd_ref[...] = jnp.zeros_like(accd_ref)

    ix = ix_ref[0]
    lane = jax.lax.broadcasted_iota(jnp.int32, (1, _BN), 1)
    base, c_row, nwin = _ranks(ix, lane, tri_ref, tris_ref, carry_ref, k)
    r_ref[0] = c_row + base

    x_bf = x_ref[...].astype(jnp.bfloat16)
    logit = jnp.dot(x_bf, wg_ref[...],
                    preferred_element_type=jnp.float32)
    e128 = jnp.exp(logit)
    e128b = e128.astype(jnp.bfloat16)
    vals = x_bf * e128b[:, 0:1]

    @pl.when(nwin <= _WF)
    def _():
        oh = _onehot_t(c_row, _WF)
        accx_ref[pl.ds(base, _WF), :] += jnp.dot(oh, vals, preferred_element_type=jnp.float32)
        accd_ref[pl.ds(base, _WF), :] += jnp.dot(oh, e128b, preferred_element_type=jnp.float32)

    @pl.when(nwin > _WF)
    def _():
        oh = _onehot_t(c_row, _BW)
        accx_ref[pl.ds(base, _BW), :] += jnp.dot(oh, vals, preferred_element_type=jnp.float32)
        accd_ref[pl.ds(base, _BW), :] += jnp.dot(oh, e128b, preferred_element_type=jnp.float32)


def _mlp_kernel(accx_ref, accd_ref, wf_ref, bf_ref, wh_ref, bh_ref, hy_ref):
    d = accd_ref[:, 0:1]
    t = accx_ref[...] / jnp.where(d > 0, d, 1.0)
    y = jax.lax.dot_general(t, wf_ref[...], (((1,), (1,)), ((), ())),
                            preferred_element_type=jnp.float32) + bf_ref[...]
    hy_ref[...] = jax.lax.dot_general(y, wh_ref[...], (((1,), (1,)), ((), ())),
                                      preferred_element_type=jnp.float32) + bh_ref[...]


def _expand_kernel(r_ref, hy_ref, out_ref):
    r = r_ref[0]
    lane = jax.lax.broadcasted_iota(jnp.int32, (1, _BN), 1)
    r0 = jnp.sum(jnp.where(lane == 0, r, 0))
    rlast = jnp.sum(jnp.where(lane == _BN - 1, r, 0))
    base = pl.multiple_of((r0 // 8) * 8, 8)
    c_row = r - base
    nwin = rlast - base + 1

    @pl.when(nwin <= _WF)
    def _():
        win = hy_ref[pl.ds(base, _WF), :].astype(jnp.bfloat16)
        out_ref[...] = jax.lax.dot_general(
            _onehot_t(c_row, _WF), win, (((0,), (0,)), ((), ())),
            preferred_element_type=jnp.float32)

    @pl.when(nwin > _WF)
    def _():
        win = hy_ref[pl.ds(base, _BW), :].astype(jnp.bfloat16)
        out_ref[...] = jax.lax.dot_general(
            _onehot_t(c_row, _BW), win, (((0,), (0,)), ((), ())),
            preferred_element_type=jnp.float32)


def kernel(x, ix, Wf, bf, Wg, bg, Wh, bh):
    x2 = x.reshape(_N, _D)
    ix3 = ix.astype(jnp.int32).reshape(_NB, 1, _BN)
    wg128 = jnp.broadcast_to(Wg.reshape(_D, 1), (_D, 128)).astype(jnp.bfloat16)
    tri = jnp.asarray(_TRI).astype(jnp.bfloat16)
    tris = jnp.asarray(_TRIS).astype(jnp.bfloat16)

    accx, accd, rks = pl.pallas_call(
        _seg_accum_kernel,
        grid=(_NB,),
        in_specs=[
            pl.BlockSpec((1, 1, _BN), lambda k: (k, 0, 0)),
            pl.BlockSpec((_BN, _D), lambda k: (k, 0)),
            pl.BlockSpec((_D, 128), lambda k: (0, 0)),
            pl.BlockSpec((128, 128), lambda k: (0, 0)),
            pl.BlockSpec((16, 16), lambda k: (0, 0)),
        ],
        out_specs=[
            pl.BlockSpec((_SPAD, _D), lambda k: (0, 0)),
            pl.BlockSpec((_SPAD, 128), lambda k: (0, 0)),
            pl.BlockSpec((1, 1, _BN), lambda k: (k, 0, 0)),
        ],
        out_shape=[
            jax.ShapeDtypeStruct((_SPAD, _D), jnp.float32),
            jax.ShapeDtypeStruct((_SPAD, 128), jnp.float32),
            jax.ShapeDtypeStruct((_NB, 1, _BN), jnp.int32),
        ],
        scratch_shapes=[pltpu.SMEM((2,), jnp.int32)],
    )(ix3, x2, wg128, tri, tris)

    hy = pl.pallas_call(
        _mlp_kernel,
        grid=(_SPAD // _BM,),
        in_specs=[
            pl.BlockSpec((_BM, _D), lambda k: (k, 0)),
            pl.BlockSpec((_BM, 128), lambda k: (k, 0)),
            pl.BlockSpec((_D, _D), lambda k: (0, 0)),
            pl.BlockSpec((1, _D), lambda k: (0, 0)),
            pl.BlockSpec((_D, _D), lambda k: (0, 0)),
            pl.BlockSpec((1, _D), lambda k: (0, 0)),
        ],
        out_specs=pl.BlockSpec((_BM, _D), lambda k: (k, 0)),
        out_shape=jax.ShapeDtypeStruct((_SPAD, _D), jnp.float32),
    )(accx, accd, Wf, bf.reshape(1, _D), Wh, bh.reshape(1, _D))

    out = pl.pallas_call(
        _expand_kernel,
        grid=(_NB,),
        in_specs=[
            pl.BlockSpec((1, 1, _BN), lambda k: (k, 0, 0)),
            pl.BlockSpec((_SPAD, _D), lambda k: (0, 0)),
        ],
        out_specs=pl.BlockSpec((_BN, _D), lambda k: (k, 0)),
        out_shape=jax.ShapeDtypeStruct((_N, _D), jnp.float32),
    )(rks, hy)

    return out.reshape(1, _N, _D)
